# Initial kernel scaffold; baseline (speedup 1.0000x reference)
import jax
import jax.numpy as jnp
from jax.experimental import pallas as pl


def kernel(x, edge_index, W1, b1, W2, b2):
    raise NotImplementedError("write your pallas kernel here")


# speedup vs baseline: 8.6016x; 8.6016x over previous
import functools

import jax
import jax.numpy as jnp
from jax import lax
from jax.experimental import pallas as pl
from jax.experimental.pallas import tpu as pltpu
from jax.experimental.pallas import tpu_sc as plsc

N = 10000
E = 320000
D = 128
NC = 2
NS = 16
NW = NC * NS
CH = 128
NCHUNK = 80
GRP = 16
EPAD = NW * NCHUNK * CH
TRASH = 10200
RPAD = 10240
RPT = RPAD // NS

_mesh = plsc.VectorSubcoreMesh(core_axis_name="c", subcore_axis_name="s")


@functools.partial(
    pl.kernel,
    out_type=jax.ShapeDtypeStruct((NC * RPAD, D), jnp.float32),
    mesh=_mesh,
    scratch_types=[
        pltpu.VMEM((NCHUNK, CH), jnp.int32),
        pltpu.VMEM((CH, D), jnp.float32),
        pltpu.VMEM((CH, D), jnp.float32),
        pltpu.VMEM_SHARED((RPAD, D), jnp.float32),
        pltpu.SemaphoreType.DMA,
        pltpu.SemaphoreType.DMA,
    ],
)
def _deg_kernel(dst_hbm, out_hbm, dst_v, ones_v, zb_v, acc, sem0, sem1):
    c = lax.axis_index("c")
    s = lax.axis_index("s")
    w = s * NC + c

    def _fill(i, _):
        ones_v[i // 8, pl.ds((i % 8) * 16, 16)] = jnp.ones((16,), jnp.float32)
        return 0
    lax.fori_loop(0, CH * 8, _fill, 0)

    def _zero(i, _):
        zb_v[i // 8, pl.ds((i % 8) * 16, 16)] = jnp.zeros((16,), jnp.float32)
        return 0
    lax.fori_loop(0, CH * 8, _zero, 0)
    for q in range(RPT // CH):
        pltpu.sync_copy(zb_v, acc.at[pl.ds(s * RPT + q * CH, CH)])
    plsc.subcore_barrier()

    pltpu.sync_copy(dst_hbm.at[pl.ds(w * NCHUNK, NCHUNK)], dst_v)

    def _step(j, _):
        pltpu.sync_copy(ones_v, acc.at[dst_v.at[j]], add=True)
        return 0
    lax.fori_loop(0, NCHUNK, _step, 0)

    plsc.subcore_barrier()
    obase = c * RPAD + s * RPT
    for q in range(RPT // CH):
        buf, sem = (ones_v, sem0) if q % 2 == 0 else (zb_v, sem1)
        if q >= 2:
            pltpu.make_async_copy(
                buf, out_hbm.at[pl.ds(obase + (q - 2) * CH, CH)], sem).wait()
        pltpu.sync_copy(acc.at[pl.ds(s * RPT + q * CH, CH)], buf)
        pltpu.async_copy(buf, out_hbm.at[pl.ds(obase + q * CH, CH)], sem)
    for q in range(RPT // CH - 2, RPT // CH):
        buf, sem = (ones_v, sem0) if q % 2 == 0 else (zb_v, sem1)
        pltpu.make_async_copy(
            buf, out_hbm.at[pl.ds(obase + q * CH, CH)], sem).wait()


@functools.partial(
    pl.kernel,
    out_type=jax.ShapeDtypeStruct((NC * RPAD, D), jnp.float32),
    mesh=_mesh,
    scratch_types=[
        pltpu.VMEM((GRP, CH), jnp.int32),
        pltpu.VMEM((GRP, CH), jnp.int32),
        pltpu.VMEM((CH, D), jnp.float32),
        pltpu.VMEM((CH, D), jnp.float32),
        pltpu.VMEM((40, D), jnp.float32),
        pltpu.VMEM_SHARED((RPAD, D), jnp.float32),
        pltpu.SemaphoreType.DMA,
        pltpu.SemaphoreType.DMA,
    ],
)
def _msg_kernel(xws_hbm, src_hbm, dst_hbm, out_hbm,
                src_v, dst_v, r0, r1, zb_v, acc, sem0, sem1):
    c = lax.axis_index("c")
    s = lax.axis_index("s")
    w = s * NC + c

    def _zero(i, _):
        r = i // 8
        col = (i % 8) * 16
        zb_v[r, pl.ds(col, 16)] = jnp.zeros((16,), jnp.float32)
        return 0
    lax.fori_loop(0, 40 * 8, _zero, 0)
    for q in range(RPT // 40):
        pltpu.sync_copy(zb_v, acc.at[pl.ds(s * RPT + q * 40, 40)])
    plsc.subcore_barrier()

    def _group(g, _):
        base = w * NCHUNK + g * GRP
        pltpu.sync_copy(src_hbm.at[pl.ds(base, GRP)], src_v)
        pltpu.sync_copy(dst_hbm.at[pl.ds(base, GRP)], dst_v)
        pltpu.async_copy(xws_hbm.at[src_v.at[0]], r0, sem0)

        def _step(i, _):
            j = i * 2
            pltpu.make_async_copy(xws_hbm.at[src_v.at[j]], r0, sem0).wait()
            pltpu.async_copy(xws_hbm.at[src_v.at[j + 1]], r1, sem1)
            pltpu.sync_copy(r0, acc.at[dst_v.at[j]], add=True)
            pltpu.make_async_copy(
                xws_hbm.at[src_v.at[j + 1]], r1, sem1).wait()

            @pl.when(j + 2 < GRP)
            def _():
                pltpu.async_copy(xws_hbm.at[src_v.at[j + 2]], r0, sem0)

            pltpu.sync_copy(r1, acc.at[dst_v.at[j + 1]], add=True)
            return 0
        lax.fori_loop(0, GRP // 2, _step, 0)
        return 0
    lax.fori_loop(0, NCHUNK // GRP, _group, 0)

    plsc.subcore_barrier()
    obase = c * RPAD + s * RPT
    for q in range(RPT // CH):
        buf, sem = (r0, sem0) if q % 2 == 0 else (r1, sem1)
        if q >= 2:
            pltpu.make_async_copy(
                buf, out_hbm.at[pl.ds(obase + (q - 2) * CH, CH)], sem).wait()
        pltpu.sync_copy(acc.at[pl.ds(s * RPT + q * CH, CH)], buf)
        pltpu.async_copy(buf, out_hbm.at[pl.ds(obase + q * CH, CH)], sem)
    for q in range(RPT // CH - 2, RPT // CH):
        buf, sem = (r0, sem0) if q % 2 == 0 else (r1, sem1)
        pltpu.make_async_copy(
            buf, out_hbm.at[pl.ds(obase + q * CH, CH)], sem).wait()


def _mm_body(x_ref, w_ref, o_ref):
    o_ref[...] = jnp.dot(x_ref[...], w_ref[...],
                         preferred_element_type=jnp.float32)


def _dis_body(da_ref, xw_ref, dis_ref, xws_ref):
    deg = da_ref[0] + da_ref[1] + 1.0
    dis = lax.rsqrt(deg)
    dis_ref[...] = dis
    xws_ref[...] = xw_ref[...] * dis


def _mid_body(acc_ref, xws_ref, dis_ref, b_ref, w_ref, o_ref):
    dis = dis_ref[...]
    tot = acc_ref[0] + acc_ref[1] + xws_ref[...]
    h = jnp.maximum(tot * dis + b_ref[...], 0.0)
    o_ref[...] = jnp.dot(h, w_ref[...],
                         preferred_element_type=jnp.float32) * dis


def _post_body(acc_ref, xws_ref, dis_ref, b_ref, o_ref):
    tot = acc_ref[0] + acc_ref[1] + xws_ref[...]
    o_ref[...] = tot * dis_ref[...] + b_ref[...]


_BR = 1000
_G = N // _BR

_row_spec = pl.BlockSpec((_BR, D), lambda i: (i, 0))
_acc_spec = pl.BlockSpec((NC, _BR, D), lambda i: (0, i, 0))
_dis_spec = pl.BlockSpec((_BR, 1), lambda i: (i, 0))
_w_spec = pl.BlockSpec((D, D), lambda i: (0, 0))
_b_spec = pl.BlockSpec((1, D), lambda i: (0, 0))


def _tc_matmul(x, W):
    return pl.pallas_call(
        _mm_body, grid=(_G,),
        in_specs=[_row_spec, _w_spec], out_specs=_row_spec,
        out_shape=jax.ShapeDtypeStruct((N, D), jnp.float32),
    )(x, W)


def _tc_dis(degacc, xw):
    return pl.pallas_call(
        _dis_body, grid=(_G,),
        in_specs=[pl.BlockSpec((NC, _BR, 1), lambda i: (0, i, 0)), _row_spec],
        out_specs=[_dis_spec, _row_spec],
        out_shape=[jax.ShapeDtypeStruct((N, 1), jnp.float32),
                   jax.ShapeDtypeStruct((N, D), jnp.float32)],
    )(degacc, xw)


def _tc_mid(acc, xws, dis, b, W):
    return pl.pallas_call(
        _mid_body, grid=(_G,),
        in_specs=[_acc_spec, _row_spec, _dis_spec, _b_spec, _w_spec],
        out_specs=_row_spec,
        out_shape=jax.ShapeDtypeStruct((N, D), jnp.float32),
    )(acc, xws, dis, b, W)


def _tc_post(acc, xws, dis, b):
    return pl.pallas_call(
        _post_body, grid=(_G,),
        in_specs=[_acc_spec, _row_spec, _dis_spec, _b_spec],
        out_specs=_row_spec,
        out_shape=jax.ShapeDtypeStruct((N, D), jnp.float32),
    )(acc, xws, dis, b)


def kernel(x, edge_index, W1, b1, W2, b2):
    npad = EPAD - E
    src = jnp.concatenate(
        [edge_index[0], jnp.zeros((npad,), jnp.int32)]).reshape(-1, CH)
    dst = jnp.concatenate(
        [edge_index[1], jnp.full((npad,), TRASH, jnp.int32)]).reshape(-1, CH)
    b1r = b1.reshape(1, D)
    b2r = b2.reshape(1, D)

    degacc = _deg_kernel(dst)[:, 0:1]
    xw1 = _tc_matmul(x, W1)
    dis, xws1 = _tc_dis(degacc.reshape(NC, RPAD, 1), xw1)
    acc1 = _msg_kernel(xws1, src, dst).reshape(NC, RPAD, D)
    xws2 = _tc_mid(acc1, xws1, dis, b1r, W2)
    acc2 = _msg_kernel(xws2, src, dst).reshape(NC, RPAD, D)
    return _tc_post(acc2, xws2, dis, b2r)


# speedup vs baseline: 9.2964x; 1.0808x over previous
import functools

import jax
import jax.numpy as jnp
from jax import lax
from jax.experimental import pallas as pl
from jax.experimental.pallas import tpu as pltpu
from jax.experimental.pallas import tpu_sc as plsc

N = 10000
E = 320000
D = 128
NC = 2
NS = 16
NW = NC * NS
CH = 128
NCHUNK = 80
MCH = 64
MNCHUNK = 160
GRPC = 40
EPAD = NW * NCHUNK * CH
TRASH = 10200
RPAD = 10240
RPT = RPAD // NS

_mesh = plsc.VectorSubcoreMesh(core_axis_name="c", subcore_axis_name="s")


@functools.partial(
    pl.kernel,
    out_type=jax.ShapeDtypeStruct((NC * RPAD, D), jnp.float32),
    mesh=_mesh,
    scratch_types=[
        pltpu.VMEM((NCHUNK, CH), jnp.int32),
        pltpu.VMEM((CH, D), jnp.float32),
        pltpu.VMEM((CH, D), jnp.float32),
        pltpu.VMEM_SHARED((RPAD, D), jnp.float32),
        pltpu.SemaphoreType.DMA,
        pltpu.SemaphoreType.DMA,
    ],
)
def _deg_kernel(dst_hbm, out_hbm, dst_v, ones_v, zb_v, acc, sem0, sem1):
    c = lax.axis_index("c")
    s = lax.axis_index("s")
    w = s * NC + c

    def _fill(i, _):
        ones_v[i // 8, pl.ds((i % 8) * 16, 16)] = jnp.ones((16,), jnp.float32)
        return 0
    lax.fori_loop(0, CH * 8, _fill, 0)

    def _zero(i, _):
        zb_v[i // 8, pl.ds((i % 8) * 16, 16)] = jnp.zeros((16,), jnp.float32)
        return 0
    lax.fori_loop(0, CH * 8, _zero, 0)
    for q in range(RPT // CH):
        pltpu.sync_copy(zb_v, acc.at[pl.ds(s * RPT + q * CH, CH)])
    plsc.subcore_barrier()

    pltpu.sync_copy(dst_hbm.at[pl.ds(w * NCHUNK, NCHUNK)], dst_v)

    def _step(j, _):
        pltpu.sync_copy(ones_v, acc.at[dst_v.at[j]], add=True)
        return 0
    lax.fori_loop(0, NCHUNK, _step, 0)

    plsc.subcore_barrier()
    obase = c * RPAD + s * RPT
    for q in range(RPT // CH):
        buf, sem = (ones_v, sem0) if q % 2 == 0 else (zb_v, sem1)
        if q >= 2:
            pltpu.make_async_copy(
                buf, out_hbm.at[pl.ds(obase + (q - 2) * CH, CH)], sem).wait()
        pltpu.sync_copy(acc.at[pl.ds(s * RPT + q * CH, CH)], buf)
        pltpu.async_copy(buf, out_hbm.at[pl.ds(obase + q * CH, CH)], sem)
    for q in range(RPT // CH - 2, RPT // CH):
        buf, sem = (ones_v, sem0) if q % 2 == 0 else (zb_v, sem1)
        pltpu.make_async_copy(
            buf, out_hbm.at[pl.ds(obase + q * CH, CH)], sem).wait()


@functools.partial(
    pl.kernel,
    out_type=jax.ShapeDtypeStruct((NC * RPAD, D), jnp.float32),
    mesh=_mesh,
    scratch_types=[
        pltpu.VMEM((GRPC, MCH), jnp.int32),
        pltpu.VMEM((GRPC, MCH), jnp.int32),
        pltpu.VMEM((MCH, D), jnp.float32),
        pltpu.VMEM((MCH, D), jnp.float32),
        pltpu.VMEM((MCH, D), jnp.float32),
        pltpu.VMEM((MCH, D), jnp.float32),
        pltpu.VMEM((16, D), jnp.float32),
        pltpu.VMEM_SHARED((RPAD, D), jnp.float32),
        pltpu.SemaphoreType.DMA,
        pltpu.SemaphoreType.DMA,
        pltpu.SemaphoreType.DMA,
        pltpu.SemaphoreType.DMA,
        pltpu.SemaphoreType.DMA,
        pltpu.SemaphoreType.DMA,
        pltpu.SemaphoreType.DMA,
        pltpu.SemaphoreType.DMA,
    ],
)
def _msg_kernel(xws_hbm, src_hbm, dst_hbm, out_hbm,
                src_v, dst_v, rb0, rb1, rb2, rb3, zb_v, acc,
                gs0, gs1, gs2, gs3, ss0, ss1, ss2, ss3):
    c = lax.axis_index("c")
    s = lax.axis_index("s")
    w = s * NC + c
    rbs = (rb0, rb1, rb2, rb3)
    gss = (gs0, gs1, gs2, gs3)
    sss = (ss0, ss1, ss2, ss3)

    def _zero(i, _):
        zb_v[i // 8, pl.ds((i % 8) * 16, 16)] = jnp.zeros((16,), jnp.float32)
        return 0
    lax.fori_loop(0, 16 * 8, _zero, 0)
    for q in range(RPT // 16):
        pltpu.sync_copy(zb_v, acc.at[pl.ds(s * RPT + q * 16, 16)])
    plsc.subcore_barrier()

    for g in range(MNCHUNK // GRPC):
        base = w * MNCHUNK + g * GRPC
        pltpu.sync_copy(src_hbm.at[pl.ds(base, GRPC)], src_v)
        pltpu.sync_copy(dst_hbm.at[pl.ds(base, GRPC)], dst_v)
        for p in range(4):
            pltpu.async_copy(xws_hbm.at[src_v.at[p]], rbs[p], gss[p])

        def _round(r, _):
            for p in range(4):
                j = r * 4 + p
                pltpu.make_async_copy(
                    xws_hbm.at[src_v.at[j]], rbs[p], gss[p]).wait()
                pltpu.async_copy(rbs[p], acc.at[dst_v.at[j]], sss[p],
                                 add=True)
            for p in range(4):
                j = r * 4 + p
                pltpu.make_async_copy(
                    rbs[p], acc.at[dst_v.at[j]], sss[p]).wait()
                pltpu.async_copy(xws_hbm.at[src_v.at[j + 4]], rbs[p], gss[p])
            return 0
        lax.fori_loop(0, GRPC // 4 - 1, _round, 0)

        jl = GRPC - 4
        for p in range(4):
            pltpu.make_async_copy(
                xws_hbm.at[src_v.at[jl + p]], rbs[p], gss[p]).wait()
            pltpu.async_copy(rbs[p], acc.at[dst_v.at[jl + p]], sss[p],
                             add=True)
        for p in range(4):
            pltpu.make_async_copy(
                rbs[p], acc.at[dst_v.at[jl + p]], sss[p]).wait()

    plsc.subcore_barrier()
    obase = c * RPAD + s * RPT
    for q in range(RPT // MCH):
        buf, sem = rbs[q % 4], sss[q % 4]
        if q >= 4:
            pltpu.make_async_copy(
                buf, out_hbm.at[pl.ds(obase + (q - 4) * MCH, MCH)],
                sem).wait()
        pltpu.sync_copy(acc.at[pl.ds(s * RPT + q * MCH, MCH)], buf)
        pltpu.async_copy(buf, out_hbm.at[pl.ds(obase + q * MCH, MCH)], sem)
    for q in range(RPT // MCH - 4, RPT // MCH):
        buf, sem = rbs[q % 4], sss[q % 4]
        pltpu.make_async_copy(
            buf, out_hbm.at[pl.ds(obase + q * MCH, MCH)], sem).wait()


def _mm_body(x_ref, w_ref, o_ref):
    o_ref[...] = jnp.dot(x_ref[...], w_ref[...],
                         preferred_element_type=jnp.float32)


def _dis_body(da_ref, xw_ref, dis_ref, xws_ref):
    deg = da_ref[0] + da_ref[1] + 1.0
    dis = lax.rsqrt(deg)
    dis_ref[...] = dis
    xws_ref[...] = xw_ref[...] * dis


def _mid_body(acc_ref, xws_ref, dis_ref, b_ref, w_ref, o_ref):
    dis = dis_ref[...]
    tot = acc_ref[0] + acc_ref[1] + xws_ref[...]
    h = jnp.maximum(tot * dis + b_ref[...], 0.0)
    o_ref[...] = jnp.dot(h, w_ref[...],
                         preferred_element_type=jnp.float32) * dis


def _post_body(acc_ref, xws_ref, dis_ref, b_ref, o_ref):
    tot = acc_ref[0] + acc_ref[1] + xws_ref[...]
    o_ref[...] = tot * dis_ref[...] + b_ref[...]


_BR = 1000
_G = N // _BR

_row_spec = pl.BlockSpec((_BR, D), lambda i: (i, 0))
_acc_spec = pl.BlockSpec((NC, _BR, D), lambda i: (0, i, 0))
_dis_spec = pl.BlockSpec((_BR, 1), lambda i: (i, 0))
_w_spec = pl.BlockSpec((D, D), lambda i: (0, 0))
_b_spec = pl.BlockSpec((1, D), lambda i: (0, 0))


def _tc_matmul(x, W):
    return pl.pallas_call(
        _mm_body, grid=(_G,),
        in_specs=[_row_spec, _w_spec], out_specs=_row_spec,
        out_shape=jax.ShapeDtypeStruct((N, D), jnp.float32),
    )(x, W)


def _tc_dis(degacc, xw):
    return pl.pallas_call(
        _dis_body, grid=(_G,),
        in_specs=[pl.BlockSpec((NC, _BR, 1), lambda i: (0, i, 0)), _row_spec],
        out_specs=[_dis_spec, _row_spec],
        out_shape=[jax.ShapeDtypeStruct((N, 1), jnp.float32),
                   jax.ShapeDtypeStruct((N, D), jnp.float32)],
    )(degacc, xw)


def _tc_mid(acc, xws, dis, b, W):
    return pl.pallas_call(
        _mid_body, grid=(_G,),
        in_specs=[_acc_spec, _row_spec, _dis_spec, _b_spec, _w_spec],
        out_specs=_row_spec,
        out_shape=jax.ShapeDtypeStruct((N, D), jnp.float32),
    )(acc, xws, dis, b, W)


def _tc_post(acc, xws, dis, b):
    return pl.pallas_call(
        _post_body, grid=(_G,),
        in_specs=[_acc_spec, _row_spec, _dis_spec, _b_spec],
        out_specs=_row_spec,
        out_shape=jax.ShapeDtypeStruct((N, D), jnp.float32),
    )(acc, xws, dis, b)


def kernel(x, edge_index, W1, b1, W2, b2):
    npad = EPAD - E
    src_flat = jnp.concatenate([edge_index[0], jnp.zeros((npad,), jnp.int32)])
    dst_flat = jnp.concatenate(
        [edge_index[1], jnp.full((npad,), TRASH, jnp.int32)])
    src = src_flat.reshape(-1, MCH)
    dst = dst_flat.reshape(-1, MCH)
    dst_deg = dst_flat.reshape(-1, CH)
    b1r = b1.reshape(1, D)
    b2r = b2.reshape(1, D)

    degacc = _deg_kernel(dst_deg)[:, 0:1]
    xw1 = _tc_matmul(x, W1)
    dis, xws1 = _tc_dis(degacc.reshape(NC, RPAD, 1), xw1)
    acc1 = _msg_kernel(xws1, src, dst).reshape(NC, RPAD, D)
    xws2 = _tc_mid(acc1, xws1, dis, b1r, W2)
    acc2 = _msg_kernel(xws2, src, dst).reshape(NC, RPAD, D)
    return _tc_post(acc2, xws2, dis, b2r)


# speedup vs baseline: 9.5994x; 1.0326x over previous
import functools

import jax
import jax.numpy as jnp
from jax import lax
from jax.experimental import pallas as pl
from jax.experimental.pallas import tpu as pltpu
from jax.experimental.pallas import tpu_sc as plsc

N = 10000
E = 320000
D = 128
NC = 2
NS = 16
NW = NC * NS
CH = 128
NCHUNK = 80
MCH = 64
MNCHUNK = 160
GRPC = 80
EPAD = NW * NCHUNK * CH
TRASH = 10200
RPAD = 10240
RPT = RPAD // NS

_mesh = plsc.VectorSubcoreMesh(core_axis_name="c", subcore_axis_name="s")


@functools.partial(
    pl.kernel,
    out_type=jax.ShapeDtypeStruct((NC * RPAD, D), jnp.float32),
    mesh=_mesh,
    scratch_types=[
        pltpu.VMEM((NCHUNK, CH), jnp.int32),
        pltpu.VMEM((CH, D), jnp.float32),
        pltpu.VMEM((CH, D), jnp.float32),
        pltpu.VMEM_SHARED((RPAD, D), jnp.float32),
        pltpu.SemaphoreType.DMA,
        pltpu.SemaphoreType.DMA,
    ],
)
def _deg_kernel(dst_hbm, out_hbm, dst_v, ones_v, zb_v, acc, sem0, sem1):
    c = lax.axis_index("c")
    s = lax.axis_index("s")
    w = s * NC + c

    def _fill(i, _):
        ones_v[i // 8, pl.ds((i % 8) * 16, 16)] = jnp.ones((16,), jnp.float32)
        return 0
    lax.fori_loop(0, CH * 8, _fill, 0)

    def _zero(i, _):
        zb_v[i // 8, pl.ds((i % 8) * 16, 16)] = jnp.zeros((16,), jnp.float32)
        return 0
    lax.fori_loop(0, CH * 8, _zero, 0)
    for q in range(RPT // CH):
        pltpu.sync_copy(zb_v, acc.at[pl.ds(s * RPT + q * CH, CH)])
    plsc.subcore_barrier()

    pltpu.sync_copy(dst_hbm.at[pl.ds(w * NCHUNK, NCHUNK)], dst_v)

    def _step(j, _):
        pltpu.sync_copy(ones_v, acc.at[dst_v.at[j]], add=True)
        return 0
    lax.fori_loop(0, NCHUNK, _step, 0)

    plsc.subcore_barrier()
    obase = c * RPAD + s * RPT
    for q in range(RPT // CH):
        buf, sem = (ones_v, sem0) if q % 2 == 0 else (zb_v, sem1)
        if q >= 2:
            pltpu.make_async_copy(
                buf, out_hbm.at[pl.ds(obase + (q - 2) * CH, CH)], sem).wait()
        pltpu.sync_copy(acc.at[pl.ds(s * RPT + q * CH, CH)], buf)
        pltpu.async_copy(buf, out_hbm.at[pl.ds(obase + q * CH, CH)], sem)
    for q in range(RPT // CH - 2, RPT // CH):
        buf, sem = (ones_v, sem0) if q % 2 == 0 else (zb_v, sem1)
        pltpu.make_async_copy(
            buf, out_hbm.at[pl.ds(obase + q * CH, CH)], sem).wait()


@functools.partial(
    pl.kernel,
    out_type=jax.ShapeDtypeStruct((NC * RPAD, D), jnp.float32),
    mesh=_mesh,
    scratch_types=[
        pltpu.VMEM((GRPC, MCH), jnp.int32),
        pltpu.VMEM((MCH,), jnp.int32),
        pltpu.VMEM((MCH,), jnp.int32),
        pltpu.VMEM((MCH,), jnp.int32),
        pltpu.VMEM((MCH,), jnp.int32),
        pltpu.VMEM((MCH,), jnp.int32),
        pltpu.VMEM((MCH,), jnp.int32),
        pltpu.VMEM((MCH,), jnp.int32),
        pltpu.VMEM((MCH,), jnp.int32),
        pltpu.VMEM((MCH, D), jnp.float32),
        pltpu.VMEM((MCH, D), jnp.float32),
        pltpu.VMEM((MCH, D), jnp.float32),
        pltpu.VMEM((MCH, D), jnp.float32),
        pltpu.VMEM_SHARED((RPAD, D), jnp.float32),
        pltpu.SemaphoreType.DMA,
        pltpu.SemaphoreType.DMA,
        pltpu.SemaphoreType.DMA,
        pltpu.SemaphoreType.DMA,
        pltpu.SemaphoreType.DMA,
        pltpu.SemaphoreType.DMA,
        pltpu.SemaphoreType.DMA,
        pltpu.SemaphoreType.DMA,
    ],
)
def _msg_kernel(xws_hbm, pk_hbm, out_hbm,
                pk_v, si0, si1, si2, si3, di0, di1, di2, di3,
                rb0, rb1, rb2, rb3, acc,
                gs0, gs1, gs2, gs3, ss0, ss1, ss2, ss3):
    c = lax.axis_index("c")
    s = lax.axis_index("s")
    w = s * NC + c
    sis = (si0, si1, si2, si3)
    dis_ = (di0, di1, di2, di3)
    rbs = (rb0, rb1, rb2, rb3)
    gss = (gs0, gs1, gs2, gs3)
    sss = (ss0, ss1, ss2, ss3)

    def _zero(i, _):
        rb0[i // 8, pl.ds((i % 8) * 16, 16)] = jnp.zeros((16,), jnp.float32)
        return 0
    lax.fori_loop(0, MCH * 8, _zero, 0)
    for q in range(RPT // MCH):
        pltpu.sync_copy(rb0, acc.at[pl.ds(s * RPT + q * MCH, MCH)])
    plsc.subcore_barrier()

    def _unpack(jrow, p):
        for u in range(MCH // 16):
            v = pk_v[jrow, pl.ds(u * 16, 16)]
            sis[p][pl.ds(u * 16, 16)] = lax.bitwise_and(v, 16383)
            dis_[p][pl.ds(u * 16, 16)] = lax.shift_right_logical(v, 14)

    for g in range(MNCHUNK // GRPC):
        pltpu.sync_copy(pk_hbm.at[pl.ds(w * MNCHUNK + g * GRPC, GRPC)], pk_v)
        for p in range(4):
            _unpack(p, p)
            pltpu.async_copy(xws_hbm.at[sis[p]], rbs[p], gss[p])

        def _round(r, _):
            for p in range(4):
                pltpu.make_async_copy(
                    xws_hbm.at[sis[p]], rbs[p], gss[p]).wait()
                pltpu.async_copy(rbs[p], acc.at[dis_[p]], sss[p], add=True)
            for p in range(4):
                pltpu.make_async_copy(
                    rbs[p], acc.at[dis_[p]], sss[p]).wait()
                _unpack(r * 4 + p + 4, p)
                pltpu.async_copy(xws_hbm.at[sis[p]], rbs[p], gss[p])
            return 0
        lax.fori_loop(0, GRPC // 4 - 1, _round, 0)

        for p in range(4):
            pltpu.make_async_copy(xws_hbm.at[sis[p]], rbs[p], gss[p]).wait()
            pltpu.async_copy(rbs[p], acc.at[dis_[p]], sss[p], add=True)
        for p in range(4):
            pltpu.make_async_copy(rbs[p], acc.at[dis_[p]], sss[p]).wait()

    plsc.subcore_barrier()
    obase = c * RPAD + s * RPT
    for q in range(RPT // MCH):
        buf, sem = rbs[q % 4], sss[q % 4]
        if q >= 4:
            pltpu.make_async_copy(
                buf, out_hbm.at[pl.ds(obase + (q - 4) * MCH, MCH)],
                sem).wait()
        pltpu.sync_copy(acc.at[pl.ds(s * RPT + q * MCH, MCH)], buf)
        pltpu.async_copy(buf, out_hbm.at[pl.ds(obase + q * MCH, MCH)], sem)
    for q in range(RPT // MCH - 4, RPT // MCH):
        buf, sem = rbs[q % 4], sss[q % 4]
        pltpu.make_async_copy(
            buf, out_hbm.at[pl.ds(obase + q * MCH, MCH)], sem).wait()


def _mm_body(x_ref, w_ref, o_ref):
    o_ref[...] = jnp.dot(x_ref[...], w_ref[...],
                         preferred_element_type=jnp.float32)


def _dis_body(da_ref, xw_ref, dis_ref, xws_ref):
    deg = da_ref[0] + da_ref[1] + 1.0
    dis = lax.rsqrt(deg)
    dis_ref[...] = dis
    xws_ref[...] = xw_ref[...] * dis


def _mid_body(acc_ref, xws_ref, dis_ref, b_ref, w_ref, o_ref):
    dis = dis_ref[...]
    tot = acc_ref[0] + acc_ref[1] + xws_ref[...]
    h = jnp.maximum(tot * dis + b_ref[...], 0.0)
    o_ref[...] = jnp.dot(h, w_ref[...],
                         preferred_element_type=jnp.float32) * dis


def _post_body(acc_ref, xws_ref, dis_ref, b_ref, o_ref):
    tot = acc_ref[0] + acc_ref[1] + xws_ref[...]
    o_ref[...] = tot * dis_ref[...] + b_ref[...]


_BR = 1000
_G = N // _BR

_row_spec = pl.BlockSpec((_BR, D), lambda i: (i, 0))
_acc_spec = pl.BlockSpec((NC, _BR, D), lambda i: (0, i, 0))
_dis_spec = pl.BlockSpec((_BR, 1), lambda i: (i, 0))
_w_spec = pl.BlockSpec((D, D), lambda i: (0, 0))
_b_spec = pl.BlockSpec((1, D), lambda i: (0, 0))


def _tc_matmul(x, W):
    return pl.pallas_call(
        _mm_body, grid=(_G,),
        in_specs=[_row_spec, _w_spec], out_specs=_row_spec,
        out_shape=jax.ShapeDtypeStruct((N, D), jnp.float32),
    )(x, W)


def _tc_dis(degacc, xw):
    return pl.pallas_call(
        _dis_body, grid=(_G,),
        in_specs=[pl.BlockSpec((NC, _BR, 1), lambda i: (0, i, 0)), _row_spec],
        out_specs=[_dis_spec, _row_spec],
        out_shape=[jax.ShapeDtypeStruct((N, 1), jnp.float32),
                   jax.ShapeDtypeStruct((N, D), jnp.float32)],
    )(degacc, xw)


def _tc_mid(acc, xws, dis, b, W):
    return pl.pallas_call(
        _mid_body, grid=(_G,),
        in_specs=[_acc_spec, _row_spec, _dis_spec, _b_spec, _w_spec],
        out_specs=_row_spec,
        out_shape=jax.ShapeDtypeStruct((N, D), jnp.float32),
    )(acc, xws, dis, b, W)


def _tc_post(acc, xws, dis, b):
    return pl.pallas_call(
        _post_body, grid=(_G,),
        in_specs=[_acc_spec, _row_spec, _dis_spec, _b_spec],
        out_specs=_row_spec,
        out_shape=jax.ShapeDtypeStruct((N, D), jnp.float32),
    )(acc, xws, dis, b)


def kernel(x, edge_index, W1, b1, W2, b2):
    npad = EPAD - E
    src_flat = jnp.concatenate([edge_index[0], jnp.zeros((npad,), jnp.int32)])
    dst_flat = jnp.concatenate(
        [edge_index[1], jnp.full((npad,), TRASH, jnp.int32)])
    packed = (src_flat | (dst_flat << 14)).reshape(-1, MCH)
    dst_deg = dst_flat.reshape(-1, CH)
    b1r = b1.reshape(1, D)
    b2r = b2.reshape(1, D)

    degacc = _deg_kernel(dst_deg)[:, 0:1]
    xw1 = _tc_matmul(x, W1)
    dis, xws1 = _tc_dis(degacc.reshape(NC, RPAD, 1), xw1)
    acc1 = _msg_kernel(xws1, packed).reshape(NC, RPAD, D)
    xws2 = _tc_mid(acc1, xws1, dis, b1r, W2)
    acc2 = _msg_kernel(xws2, packed).reshape(NC, RPAD, D)
    return _tc_post(acc2, xws2, dis, b2r)
